# Initial kernel scaffold; baseline (speedup 1.0000x reference)
import jax
import jax.numpy as jnp
from jax.experimental import pallas as pl


def kernel(sentence, weight):
    raise NotImplementedError("write your pallas kernel here")


# speedup vs baseline: 2.8062x; 2.8062x over previous
import functools

import jax
import jax.numpy as jnp
from jax import lax
from jax.experimental import pallas as pl
from jax.experimental.pallas import tpu as pltpu
from jax.experimental.pallas import tpu_sc as plsc

B = 16384
L = 50
D = 32
LANES = 16
NC = 2
NS = 16
NW = NC * NS
BAGS_PER_W = B // NW
C = 16
NCHUNK = BAGS_PER_W // C
ROWS_PER_CHUNK = C * L
LPAD = 64


def _bag_compute(rows_ref, idxp_ref, out_ref, j):
    base = j * L
    acc0 = jnp.zeros((LANES,), jnp.float32)
    acc1 = jnp.zeros((LANES,), jnp.float32)
    for r in range(L):
        acc0 = acc0 + rows_ref[base + r, pl.ds(0, LANES)]
        acc1 = acc1 + rows_ref[base + r, pl.ds(LANES, LANES)]
    cnt = jnp.zeros((LANES,), jnp.int32)
    for k in range(LPAD // LANES):
        idx_v = idxp_ref[j, pl.ds(k * LANES, LANES)]
        cnt = cnt + plsc.all_reduce_population_count(idx_v != 0)
    denom = jnp.maximum(cnt.astype(jnp.float32), 1.0)
    out_ref[j, pl.ds(0, LANES)] = acc0 / denom
    out_ref[j, pl.ds(LANES, LANES)] = acc1 / denom


def _emb_bag_kernel(idx_flat, idx_pad, table, out,
                    idxg0, idxg1, idxp0, idxp1, rows0, rows1,
                    outb0, outb1, gsem0, gsem1, osem0, osem1):
    wid = lax.axis_index("s") * NC + lax.axis_index("c")
    w_bag0 = wid * BAGS_PER_W

    idxg = (idxg0, idxg1)
    idxp = (idxp0, idxp1)
    rows = (rows0, rows1)
    outb = (outb0, outb1)
    gsem = (gsem0, gsem1)
    osem = (osem0, osem1)

    def load_chunk(chunk, b):
        bag0 = w_bag0 + chunk * C
        pltpu.sync_copy(idx_flat.at[pl.ds(bag0 * L, ROWS_PER_CHUNK)], idxg[b])
        pltpu.sync_copy(idx_pad.at[pl.ds(bag0, C)], idxp[b])
        pltpu.async_copy(table.at[idxg[b]], rows[b], gsem[b])

    for b in range(2):
        load_chunk(b, b)

    @pl.loop(0, NCHUNK, step=2)
    def _chunks(g):
        for b in range(2):
            chunk = g + b
            bag0 = w_bag0 + chunk * C
            pltpu.make_async_copy(table.at[idxg[b]], rows[b], gsem[b]).wait()

            @pl.when(chunk >= 2)
            def _():
                pltpu.make_async_copy(
                    outb[b], out.at[pl.ds(bag0, C)], osem[b]).wait()

            @pl.loop(0, C)
            def _bags(j):
                _bag_compute(rows[b], idxp[b], outb[b], j)

            pltpu.async_copy(outb[b], out.at[pl.ds(bag0, C)], osem[b])

            @pl.when(chunk + 2 < NCHUNK)
            def _():
                load_chunk(chunk + 2, b)

    for b in range(2):
        pltpu.make_async_copy(
            outb[b], out.at[pl.ds(0, C)], osem[b]).wait()


@jax.jit
def _emb_bag(idx_flat, idx_pad, table):
    mesh = plsc.VectorSubcoreMesh(core_axis_name="c", subcore_axis_name="s")
    return pl.kernel(
        _emb_bag_kernel,
        out_type=jax.ShapeDtypeStruct((B, D), jnp.float32),
        mesh=mesh,
        compiler_params=pltpu.CompilerParams(
            needs_layout_passes=False, use_tc_tiling_on_sc=False),
        scratch_types=[
            pltpu.VMEM((ROWS_PER_CHUNK,), jnp.int32),
            pltpu.VMEM((ROWS_PER_CHUNK,), jnp.int32),
            pltpu.VMEM((C, LPAD), jnp.int32),
            pltpu.VMEM((C, LPAD), jnp.int32),
            pltpu.VMEM((ROWS_PER_CHUNK, D), jnp.float32),
            pltpu.VMEM((ROWS_PER_CHUNK, D), jnp.float32),
            pltpu.VMEM((C, D), jnp.float32),
            pltpu.VMEM((C, D), jnp.float32),
            pltpu.SemaphoreType.DMA,
            pltpu.SemaphoreType.DMA,
            pltpu.SemaphoreType.DMA,
            pltpu.SemaphoreType.DMA,
        ],
    )(idx_flat, idx_pad, table)


def kernel(sentence, weight):
    idx = sentence.astype(jnp.int32)
    idx_flat = idx.reshape(-1)
    idx_pad = jnp.pad(idx, ((0, 0), (0, LPAD - L)))
    return _emb_bag(idx_flat, idx_pad, weight)


# speedup vs baseline: 2.9154x; 1.0389x over previous
import functools

import jax
import jax.numpy as jnp
from jax import lax
from jax.experimental import pallas as pl
from jax.experimental.pallas import tpu as pltpu
from jax.experimental.pallas import tpu_sc as plsc

B = 16384
L = 50
D = 32
LANES = 16
NC = 2
NS = 16
NW = NC * NS
BAGS_PER_W = B // NW
C = 16
NCHUNK = BAGS_PER_W // C
ROWS_PER_CHUNK = C * L
LPAD = 64


def _bag_compute(rows_ref, idxg_ref, out_ref, j):
    base = j * L
    acc0 = jnp.zeros((LANES,), jnp.float32)
    acc1 = jnp.zeros((LANES,), jnp.float32)
    for r in range(L):
        acc0 = acc0 + rows_ref[base + r, pl.ds(0, LANES)]
        acc1 = acc1 + rows_ref[base + r, pl.ds(LANES, LANES)]
    cnt = jnp.zeros((LANES,), jnp.int32)
    for off in (0, LANES, 2 * LANES):
        idx_v = idxg_ref[pl.ds(base + off, LANES)]
        cnt = cnt + plsc.all_reduce_population_count(idx_v != 0)
    tail = idxg_ref[pl.ds(base + L - LANES, LANES)]
    lane = lax.iota(jnp.int32, LANES)
    cnt = cnt + plsc.all_reduce_population_count((tail != 0) & (lane >= 14))
    denom = jnp.maximum(cnt.astype(jnp.float32), 1.0)
    out_ref[j, pl.ds(0, LANES)] = acc0 / denom
    out_ref[j, pl.ds(LANES, LANES)] = acc1 / denom


def _emb_bag_kernel(idx_flat, table, out,
                    idxg0, idxg1, rows0, rows1,
                    outb0, outb1, gsem0, gsem1, osem0, osem1):
    wid = lax.axis_index("s") * NC + lax.axis_index("c")
    w_bag0 = wid * BAGS_PER_W

    idxg = (idxg0, idxg1)
    rows = (rows0, rows1)
    outb = (outb0, outb1)
    gsem = (gsem0, gsem1)
    osem = (osem0, osem1)

    def load_chunk(chunk, b):
        bag0 = w_bag0 + chunk * C
        pltpu.sync_copy(idx_flat.at[pl.ds(bag0 * L, ROWS_PER_CHUNK)], idxg[b])
        pltpu.async_copy(table.at[idxg[b]], rows[b], gsem[b])

    for b in range(2):
        load_chunk(b, b)

    @pl.loop(0, NCHUNK, step=2)
    def _chunks(g):
        for b in range(2):
            chunk = g + b
            bag0 = w_bag0 + chunk * C
            pltpu.make_async_copy(table.at[idxg[b]], rows[b], gsem[b]).wait()

            @pl.when(chunk >= 2)
            def _():
                pltpu.make_async_copy(
                    outb[b], out.at[pl.ds(bag0, C)], osem[b]).wait()

            @pl.loop(0, C)
            def _bags(j):
                _bag_compute(rows[b], idxg[b], outb[b], j)

            pltpu.async_copy(outb[b], out.at[pl.ds(bag0, C)], osem[b])

            @pl.when(chunk + 2 < NCHUNK)
            def _():
                load_chunk(chunk + 2, b)

    for b in range(2):
        pltpu.make_async_copy(
            outb[b], out.at[pl.ds(0, C)], osem[b]).wait()


@jax.jit
def _emb_bag(idx_flat, table):
    mesh = plsc.VectorSubcoreMesh(core_axis_name="c", subcore_axis_name="s")
    return pl.kernel(
        _emb_bag_kernel,
        out_type=jax.ShapeDtypeStruct((B, D), jnp.float32),
        mesh=mesh,
        compiler_params=pltpu.CompilerParams(
            needs_layout_passes=False, use_tc_tiling_on_sc=False),
        scratch_types=[
            pltpu.VMEM((ROWS_PER_CHUNK,), jnp.int32),
            pltpu.VMEM((ROWS_PER_CHUNK,), jnp.int32),
            pltpu.VMEM((ROWS_PER_CHUNK, D), jnp.float32),
            pltpu.VMEM((ROWS_PER_CHUNK, D), jnp.float32),
            pltpu.VMEM((C, D), jnp.float32),
            pltpu.VMEM((C, D), jnp.float32),
            pltpu.SemaphoreType.DMA,
            pltpu.SemaphoreType.DMA,
            pltpu.SemaphoreType.DMA,
            pltpu.SemaphoreType.DMA,
        ],
    )(idx_flat, table)


def kernel(sentence, weight):
    idx_flat = sentence.astype(jnp.int32).reshape(-1)
    return _emb_bag(idx_flat, weight)


# speedup vs baseline: 2.9188x; 1.0012x over previous
import jax
import jax.numpy as jnp
from jax import lax
from jax.experimental import pallas as pl
from jax.experimental.pallas import tpu as pltpu
from jax.experimental.pallas import tpu_sc as plsc

B = 16384
L = 50
D = 32
LANES = 16
NC = 2
NS = 16
NW = NC * NS
BAGS_PER_W = B // NW
C = 16
NCHUNK = BAGS_PER_W // C
ROWS_PER_CHUNK = C * L
GCHUNKS = B // C


def _bag_compute(rows_ref, idx_ref, out_ref, j):
    base = j * L
    acc0 = jnp.zeros((LANES,), jnp.float32)
    acc1 = jnp.zeros((LANES,), jnp.float32)
    for r in range(L):
        acc0 = acc0 + rows_ref[base + r, pl.ds(0, LANES)]
        acc1 = acc1 + rows_ref[base + r, pl.ds(LANES, LANES)]
    cnt = jnp.zeros((LANES,), jnp.int32)
    for off in (0, LANES, 2 * LANES):
        idx_v = idx_ref[pl.ds(base + off, LANES)]
        cnt = cnt + plsc.all_reduce_population_count(idx_v != 0)
    tail = idx_ref[pl.ds(base + L - LANES, LANES)]
    lane = lax.iota(jnp.int32, LANES)
    cnt = cnt + plsc.all_reduce_population_count((tail != 0) & (lane >= 14))
    denom = jnp.maximum(cnt.astype(jnp.float32), 1.0)
    out_ref[j, pl.ds(0, LANES)] = acc0 / denom
    out_ref[j, pl.ds(LANES, LANES)] = acc1 / denom


def _emb_bag_kernel(idx_rows, table, out,
                    idxf0, idxf1, rows0, rows1,
                    outb0, outb1, gsem0, gsem1, osem0, osem1):
    wid = lax.axis_index("s") * NC + lax.axis_index("c")
    w_chunk0 = wid * NCHUNK
    w_bag0 = wid * BAGS_PER_W

    idxf = (idxf0, idxf1)
    rows = (rows0, rows1)
    outb = (outb0, outb1)
    gsem = (gsem0, gsem1)
    osem = (osem0, osem1)

    def load_chunk(chunk, b):
        pltpu.sync_copy(idx_rows.at[w_chunk0 + chunk], idxf[b])
        pltpu.async_copy(table.at[idxf[b]], rows[b], gsem[b])

    for b in range(2):
        load_chunk(b, b)

    @pl.loop(0, NCHUNK, step=2)
    def _chunks(g):
        for b in range(2):
            chunk = g + b
            bag0 = w_bag0 + chunk * C
            pltpu.make_async_copy(table.at[idxf[b]], rows[b], gsem[b]).wait()

            @pl.when(chunk >= 2)
            def _():
                pltpu.make_async_copy(
                    outb[b], out.at[pl.ds(bag0, C)], osem[b]).wait()

            @pl.loop(0, C)
            def _bags(j):
                _bag_compute(rows[b], idxf[b], outb[b], j)

            pltpu.async_copy(outb[b], out.at[pl.ds(bag0, C)], osem[b])

            @pl.when(chunk + 2 < NCHUNK)
            def _():
                load_chunk(chunk + 2, b)

    for b in range(2):
        pltpu.make_async_copy(
            outb[b], out.at[pl.ds(0, C)], osem[b]).wait()


@jax.jit
def _emb_bag(idx_rows, table):
    mesh = plsc.VectorSubcoreMesh(core_axis_name="c", subcore_axis_name="s")
    return pl.kernel(
        _emb_bag_kernel,
        out_type=jax.ShapeDtypeStruct((B, D), jnp.float32),
        mesh=mesh,
        compiler_params=pltpu.CompilerParams(
            needs_layout_passes=False, use_tc_tiling_on_sc=False),
        scratch_types=[
            pltpu.VMEM((ROWS_PER_CHUNK,), jnp.int32),
            pltpu.VMEM((ROWS_PER_CHUNK,), jnp.int32),
            pltpu.VMEM((ROWS_PER_CHUNK, D), jnp.float32),
            pltpu.VMEM((ROWS_PER_CHUNK, D), jnp.float32),
            pltpu.VMEM((C, D), jnp.float32),
            pltpu.VMEM((C, D), jnp.float32),
            pltpu.SemaphoreType.DMA,
            pltpu.SemaphoreType.DMA,
            pltpu.SemaphoreType.DMA,
            pltpu.SemaphoreType.DMA,
        ],
    )(idx_rows, table)


def kernel(sentence, weight):
    idx_rows = sentence.astype(jnp.int32).reshape(GCHUNKS, ROWS_PER_CHUNK)
    return _emb_bag(idx_rows, weight)
